# Initial kernel scaffold; baseline (speedup 1.0000x reference)
import jax
import jax.numpy as jnp
from jax.experimental import pallas as pl


def kernel(x, dw1, db1, dw2, db2, codebooks, uw1, ub1, uw2, ub2):
    raise NotImplementedError("write your pallas kernel here")


# speedup vs baseline: 1.7850x; 1.7850x over previous
import functools

import jax
import jax.numpy as jnp
from jax import lax
from jax.experimental import pallas as pl
from jax.experimental.pallas import tpu as pltpu
from jax.experimental.pallas import tpu_sc as plsc

NCB = 4
NE = 8192
DS = 64
DP = 128
EMBP = NCB * DP
IN_DIM = 1024
HID = 4096
LLM = 2048
LLM4 = 8192
CC = 0.25
N_TOK = 8192

TILE_A = 256
GRID_A = N_TOK // TILE_A
NCHUNK = 4
CH = NE // NCHUNK

TILE_C = 128
GRID_C = N_TOK // TILE_C

_INV_SQRT2 = 0.7071067811865476


def _gelu_exact(v):
    return 0.5 * v * (1.0 + lax.erf(v * _INV_SQRT2))


def _mm_body(a_ref, b_ref, o_ref):
    o_ref[...] = jnp.dot(a_ref[...], b_ref[...],
                         preferred_element_type=jnp.float32)


def _down_vq_body(h_ref, dw2_ref, db2_ref, cbt_ref, cn_ref,
                  zp_ref, idx_ref, fidx_ref, ent_ref, counts_scr):
    t = pl.program_id(0)

    @pl.when(t == 0)
    def _init():
        counts_scr[...] = jnp.zeros_like(counts_scr)

    zp = jnp.dot(h_ref[...], dw2_ref[...],
                 preferred_element_type=jnp.float32)
    zp = zp + db2_ref[...]
    zp_ref[...] = zp

    idx_cols = []
    for i in range(NCB):
        s = zp[:, i * DP:(i + 1) * DP].astype(jnp.bfloat16)
        m = None
        ii = None
        for c in range(NCHUNK):
            cb_sl = cbt_ref[i, :, c * CH:(c + 1) * CH]
            dc = jnp.dot(s, cb_sl, preferred_element_type=jnp.float32)
            dc = dc + cn_ref[i, :, c * CH:(c + 1) * CH]
            lane = lax.broadcasted_iota(jnp.int32, (TILE_A, CH), 1) + c * CH
            mc = jnp.min(dc, axis=1, keepdims=True)
            ic = jnp.min(jnp.where(dc <= mc, lane, jnp.int32(2 ** 30)),
                         axis=1, keepdims=True)
            if m is None:
                m, ii = mc, ic
            else:
                upd = mc < m
                ii = jnp.where(upd, ic, ii)
                m = jnp.where(upd, mc, m)
        idx_cols.append(ii)
        for c in range(NCHUNK):
            lane = lax.broadcasted_iota(jnp.int32, (TILE_A, CH), 1) + c * CH
            cnt = jnp.sum(jnp.where(ii == lane, 1.0, 0.0),
                          axis=0, keepdims=True)
            sl = pl.ds(c * CH, CH)
            counts_scr[i:i + 1, sl] += cnt

    idx_blk = jnp.concatenate(idx_cols, axis=1)
    idx_ref[...] = idx_blk
    off = lax.broadcasted_iota(jnp.int32, (1, NCB), 1) * NE
    fidx_ref[...] = idx_blk + off

    @pl.when(t == GRID_A - 1)
    def _entropy():
        counts = counts_scr[0:NCB, :]
        p = counts * (1.0 / N_TOK)
        ent = -jnp.sum(p * jnp.log(p + 1e-10), keepdims=True)
        ent_ref[...] = ent.reshape(1, 1) * (1.0 / NCB)


def _up_body(zq_ref, zp_ref, uw1_ref, ub1_ref, uw2_ref, ub2_ref,
             xvq_ref, loss_ref, commit_ref, acc_scr):
    t = pl.program_id(0)

    @pl.when(t == 0)
    def _init():
        acc_scr[...] = jnp.zeros_like(acc_scr)

    zq = zq_ref[...]
    hu = jnp.dot(zq.astype(jnp.bfloat16), uw1_ref[...],
                 preferred_element_type=jnp.float32)
    hu = _gelu_exact(hu + ub1_ref[...])
    out = jnp.dot(hu.astype(jnp.bfloat16), uw2_ref[...],
                  preferred_element_type=jnp.float32)
    xvq_ref[...] = out + ub2_ref[...]

    diff = zq - zp_ref[...]
    acc_scr[...] += jnp.sum(diff * diff, keepdims=True).reshape(1, 1)

    @pl.when(t == GRID_C - 1)
    def _final():
        val = acc_scr[...] * (CC / (NCB * N_TOK * DS))
        loss_ref[...] = val
        commit_ref[...] = val


def _make_sc_gather():
    info = plsc.get_sparse_core_info()
    nw = info.num_cores * info.num_subcores
    rows_per_w = (N_TOK * NCB) // nw
    chunk = 128
    nch = rows_per_w // chunk
    mesh = plsc.VectorSubcoreMesh(core_axis_name="c", subcore_axis_name="s")

    @functools.partial(
        pl.kernel, mesh=mesh,
        out_type=jax.ShapeDtypeStruct((N_TOK * NCB, DP), jnp.float32),
        scratch_types=[
            pltpu.VMEM((chunk,), jnp.int32),
            pltpu.VMEM((chunk, DP), jnp.float32),
            pltpu.SemaphoreType.DMA,
        ],
    )
    def sc_gather(table_hbm, fidx_hbm, out_hbm, idx_v, rows_v, sem):
        wid = lax.axis_index("s") * info.num_cores + lax.axis_index("c")
        base = wid * rows_per_w
        for c in range(nch):
            off = base + c * chunk
            pltpu.sync_copy(fidx_hbm.at[pl.ds(off, chunk)], idx_v)
            pltpu.async_copy(table_hbm.at[idx_v], rows_v, sem).wait()
            pltpu.sync_copy(rows_v, out_hbm.at[pl.ds(off, chunk)])

    return sc_gather


def kernel(x, dw1, db1, dw2, db2, codebooks, uw1, ub1, uw2, ub2):
    xf = x.reshape(N_TOK, IN_DIM)

    xb16 = xf.astype(jnp.bfloat16)
    dw1b = dw1.astype(jnp.bfloat16)
    dw2p = jnp.pad(dw2.reshape(HID, NCB, DS), ((0, 0), (0, 0), (0, DP - DS)))
    dw2p = dw2p.reshape(HID, EMBP).astype(jnp.bfloat16)
    db2p = jnp.pad(db2.reshape(NCB, DS), ((0, 0), (0, DP - DS)))
    db2p = db2p.reshape(1, EMBP)
    cbt = jnp.pad(jnp.transpose(codebooks, (0, 2, 1)) * (-2.0),
                  ((0, 0), (0, DP - DS), (0, 0))).astype(jnp.bfloat16)
    cn = jnp.sum(codebooks * codebooks, axis=-1)[:, None, :]
    table = jnp.pad(codebooks, ((0, 0), (0, 0), (0, DP - DS)))
    table = table.reshape(NCB * NE, DP)
    uw1p = jnp.pad(uw1.reshape(NCB, DS, LLM4), ((0, 0), (0, DP - DS), (0, 0)))
    uw1p = uw1p.reshape(EMBP, LLM4).astype(jnp.bfloat16)
    uw2b = uw2.astype(jnp.bfloat16)

    hpre = pl.pallas_call(
        _mm_body,
        grid=(GRID_A,),
        in_specs=[
            pl.BlockSpec((TILE_A, IN_DIM), lambda t: (t, 0)),
            pl.BlockSpec((IN_DIM, HID), lambda t: (0, 0)),
        ],
        out_specs=pl.BlockSpec((TILE_A, HID), lambda t: (t, 0)),
        out_shape=jax.ShapeDtypeStruct((N_TOK, HID), jnp.float32),
    )(xb16, dw1b)
    h16 = jax.nn.gelu(hpre + db1, approximate=False).astype(jnp.bfloat16)

    zp, idx, fidx, ent = pl.pallas_call(
        _down_vq_body,
        grid=(GRID_A,),
        in_specs=[
            pl.BlockSpec((TILE_A, HID), lambda t: (t, 0)),
            pl.BlockSpec((HID, EMBP), lambda t: (0, 0)),
            pl.BlockSpec((1, EMBP), lambda t: (0, 0)),
            pl.BlockSpec((NCB, DP, NE), lambda t: (0, 0, 0)),
            pl.BlockSpec((NCB, 1, NE), lambda t: (0, 0, 0)),
        ],
        out_specs=[
            pl.BlockSpec((TILE_A, EMBP), lambda t: (t, 0)),
            pl.BlockSpec((TILE_A, NCB), lambda t: (t, 0)),
            pl.BlockSpec((TILE_A, NCB), lambda t: (t, 0)),
            pl.BlockSpec((1, 1), lambda t: (0, 0)),
        ],
        out_shape=[
            jax.ShapeDtypeStruct((N_TOK, EMBP), jnp.float32),
            jax.ShapeDtypeStruct((N_TOK, NCB), jnp.int32),
            jax.ShapeDtypeStruct((N_TOK, NCB), jnp.int32),
            jax.ShapeDtypeStruct((1, 1), jnp.float32),
        ],
        scratch_shapes=[pltpu.VMEM((8, NE), jnp.float32)],
    )(h16, dw2p, db2p, cbt, cn)

    zqp = _make_sc_gather()(table, fidx.reshape(N_TOK * NCB))
    zqp = zqp.reshape(N_TOK, EMBP)

    xvq, loss, commit = pl.pallas_call(
        _up_body,
        grid=(GRID_C,),
        in_specs=[
            pl.BlockSpec((TILE_C, EMBP), lambda t: (t, 0)),
            pl.BlockSpec((TILE_C, EMBP), lambda t: (t, 0)),
            pl.BlockSpec((EMBP, LLM4), lambda t: (0, 0)),
            pl.BlockSpec((1, LLM4), lambda t: (0, 0)),
            pl.BlockSpec((LLM4, LLM), lambda t: (0, 0)),
            pl.BlockSpec((1, LLM), lambda t: (0, 0)),
        ],
        out_specs=[
            pl.BlockSpec((TILE_C, LLM), lambda t: (t, 0)),
            pl.BlockSpec((1, 1), lambda t: (0, 0)),
            pl.BlockSpec((1, 1), lambda t: (0, 0)),
        ],
        out_shape=[
            jax.ShapeDtypeStruct((N_TOK, LLM), jnp.float32),
            jax.ShapeDtypeStruct((1, 1), jnp.float32),
            jax.ShapeDtypeStruct((1, 1), jnp.float32),
        ],
        scratch_shapes=[pltpu.VMEM((1, 1), jnp.float32)],
    )(zqp, zp, uw1p, ub1.reshape(1, LLM4), uw2b, ub2.reshape(1, LLM))

    Bb, Ll = x.shape[0], x.shape[1]
    return (xvq.reshape(Bb, Ll, LLM),
            idx.reshape(Bb, Ll, NCB),
            loss[0, 0], commit[0, 0], ent[0, 0])
